# baseline (speedup 1.0000x reference)
import jax
import jax.numpy as jnp
from jax.experimental import pallas as pl
from jax.experimental.pallas import tpu as pltpu

_BLK = 5000
_NBUF = 4


def _make_body(nblk):
    def body(w1_ref, adj_ref, b_ref, x_hbm, o_hbm, x_buf, o_buf, in_sem, out_sem):
        w = jnp.dot(w1_ref[...], adj_ref[...], preferred_element_type=jnp.float32)
        bias = b_ref[...]

        def in_copy(i, slot):
            return pltpu.make_async_copy(
                x_hbm.at[pl.ds(i * _BLK, _BLK), :], x_buf.at[slot], in_sem.at[slot]
            )

        def out_copy(i, slot):
            return pltpu.make_async_copy(
                o_buf.at[slot], o_hbm.at[pl.ds(i * _BLK, _BLK), :], out_sem.at[slot]
            )

        for s in range(_NBUF):
            in_copy(s, s).start()

        def step(outer, carry):
            for s in range(_NBUF):
                i = outer * _NBUF + s
                in_copy(i, s).wait()

                @pl.when(outer > 0)
                def _():
                    out_copy(i - _NBUF, s).wait()

                o_buf[s] = (
                    jnp.dot(x_buf[s], w, preferred_element_type=jnp.float32) + bias
                )
                out_copy(i, s).start()

                @pl.when(i + _NBUF < nblk)
                def _():
                    in_copy(i + _NBUF, s).start()

            return carry

        jax.lax.fori_loop(0, nblk // _NBUF, step, 0)

        for s in range(_NBUF):
            out_copy(nblk - _NBUF + s, s).wait()

    return body


def kernel(x, adj, W1, b):
    n, din = x.shape
    dout = adj.shape[1]
    assert n % (_BLK * _NBUF) == 0
    nblk = n // _BLK

    return pl.pallas_call(
        _make_body(nblk),
        in_specs=[
            pl.BlockSpec(memory_space=pltpu.VMEM),
            pl.BlockSpec(memory_space=pltpu.VMEM),
            pl.BlockSpec(memory_space=pltpu.VMEM),
            pl.BlockSpec(memory_space=pltpu.HBM),
        ],
        out_specs=pl.BlockSpec(memory_space=pltpu.HBM),
        out_shape=jax.ShapeDtypeStruct((n, dout), x.dtype),
        scratch_shapes=[
            pltpu.VMEM((_NBUF, _BLK, din), jnp.float32),
            pltpu.VMEM((_NBUF, _BLK, dout), jnp.float32),
            pltpu.SemaphoreType.DMA((_NBUF,)),
            pltpu.SemaphoreType.DMA((_NBUF,)),
        ],
    )(W1, adj, b.reshape(1, dout), x)


# speedup vs baseline: 1.0249x; 1.0249x over previous
import jax
import jax.numpy as jnp
from jax.experimental import pallas as pl
from jax.experimental.pallas import tpu as pltpu

_BLK = 2000
_NBUF = 10


def _make_body(nblk):
    def body(w1_ref, adj_ref, b_ref, x_hbm, o_hbm, x_buf, o_buf, in_sem, out_sem):
        w = jnp.dot(w1_ref[...], adj_ref[...], preferred_element_type=jnp.float32)
        bias = b_ref[...]

        def in_copy(i, slot):
            return pltpu.make_async_copy(
                x_hbm.at[pl.ds(i * _BLK, _BLK), :], x_buf.at[slot], in_sem.at[slot]
            )

        def out_copy(i, slot):
            return pltpu.make_async_copy(
                o_buf.at[slot], o_hbm.at[pl.ds(i * _BLK, _BLK), :], out_sem.at[slot]
            )

        for s in range(_NBUF):
            in_copy(s, s).start()

        def step(outer, carry):
            for s in range(_NBUF):
                i = outer * _NBUF + s
                in_copy(i, s).wait()

                @pl.when(outer > 0)
                def _():
                    out_copy(i - _NBUF, s).wait()

                o_buf[s] = (
                    jnp.dot(x_buf[s], w, preferred_element_type=jnp.float32) + bias
                )
                out_copy(i, s).start()

                @pl.when(i + _NBUF < nblk)
                def _():
                    in_copy(i + _NBUF, s).start()

            return carry

        jax.lax.fori_loop(0, nblk // _NBUF, step, 0)

        for s in range(_NBUF):
            out_copy(nblk - _NBUF + s, s).wait()

    return body


def kernel(x, adj, W1, b):
    n, din = x.shape
    dout = adj.shape[1]
    assert n % (_BLK * _NBUF) == 0
    nblk = n // _BLK

    return pl.pallas_call(
        _make_body(nblk),
        in_specs=[
            pl.BlockSpec(memory_space=pltpu.VMEM),
            pl.BlockSpec(memory_space=pltpu.VMEM),
            pl.BlockSpec(memory_space=pltpu.VMEM),
            pl.BlockSpec(memory_space=pltpu.HBM),
        ],
        out_specs=pl.BlockSpec(memory_space=pltpu.HBM),
        out_shape=jax.ShapeDtypeStruct((n, dout), x.dtype),
        scratch_shapes=[
            pltpu.VMEM((_NBUF, _BLK, din), jnp.float32),
            pltpu.VMEM((_NBUF, _BLK, dout), jnp.float32),
            pltpu.SemaphoreType.DMA((_NBUF,)),
            pltpu.SemaphoreType.DMA((_NBUF,)),
        ],
    )(W1, adj, b.reshape(1, dout), x)
